# baseline (speedup 1.0000x reference)
import functools

import jax
import jax.numpy as jnp
from jax import lax
from jax.experimental import pallas as pl
from jax.experimental.pallas import tpu as pltpu
from jax.experimental.pallas import tpu_sc as plsc

NUM_FIELDS = 26
VOCAB = 100000
EMBED_DIM = 16
BATCH = 4096
DEEP_IN = NUM_FIELDS * EMBED_DIM
H1, H2 = 128, 64

_NC, _NS, _L = 2, 16, 16
_NW = _NC * _NS
_BPW = BATCH // _NW
_NPW = _BPW * NUM_FIELDS
_CHUNK = 128
_NCHUNK = _NPW // _CHUNK


def _sc_body(cat_hbm, fm_hbm, lin_hbm,
             rows_hbm, lin_out_hbm,
             cat_v, idx_sm, rows_v, lin_v, sem):
    wid = lax.axis_index("s") * _NC + lax.axis_index("c")
    base = wid * _BPW

    pltpu.sync_copy(cat_hbm.at[pl.ds(base * NUM_FIELDS, _NPW)], cat_v)

    lanes = lax.iota(jnp.int32, _L)
    for j in range(_NPW // _L):
        pos = lanes + (_L * j)
        off = lax.rem(pos, NUM_FIELDS) * VOCAB
        idx_sm[pl.ds(_L * j, _L)] = cat_v[pl.ds(_L * j, _L)] + off

    copies = []
    for cch in range(_NCHUNK):
        idx = idx_sm.at[pl.ds(cch * _CHUNK, _CHUNK)]
        copies.append(pltpu.async_copy(
            fm_hbm.at[idx],
            rows_v.at[pl.ds(cch * _CHUNK, _CHUNK), :], sem))
        copies.append(pltpu.async_copy(
            lin_hbm.at[idx],
            lin_v.at[pl.ds(cch * _CHUNK, _CHUNK)], sem))
    for cp in copies:
        cp.wait()

    pltpu.sync_copy(rows_v, rows_hbm.at[pl.ds(base * NUM_FIELDS, _NPW), :])
    pltpu.sync_copy(lin_v, lin_out_hbm.at[pl.ds(base * NUM_FIELDS, _NPW)])


_sc_gather = functools.partial(
    pl.kernel,
    out_type=[
        jax.ShapeDtypeStruct((BATCH * NUM_FIELDS, EMBED_DIM), jnp.float32),
        jax.ShapeDtypeStruct((BATCH * NUM_FIELDS,), jnp.float32),
    ],
    mesh=plsc.VectorSubcoreMesh(core_axis_name="c", subcore_axis_name="s"),
    scratch_types=[
        pltpu.VMEM((_NPW,), jnp.int32),
        pltpu.VMEM((_NPW,), jnp.int32),
        pltpu.VMEM((_NPW, EMBED_DIM), jnp.float32),
        pltpu.VMEM((_NPW,), jnp.float32),
        pltpu.SemaphoreType.DMA,
    ],
    compiler_params=pltpu.CompilerParams(use_tc_tiling_on_sc=False),
)(_sc_body)


_BLK = 512


def _tc_body(x_ref, le_ref, w1_ref, b1_ref, w2_ref, b2_ref, w3_ref, bias_ref,
             o_ref):
    x = x_ref[...]
    lin = jnp.sum(le_ref[...], axis=1, keepdims=True)
    ii = lax.broadcasted_iota(jnp.int32, (DEEP_IN, EMBED_DIM), 0)
    jj = lax.broadcasted_iota(jnp.int32, (DEEP_IN, EMBED_DIM), 1)
    sel = (lax.rem(ii, EMBED_DIM) == jj).astype(jnp.float32)
    s = jnp.dot(x, sel, preferred_element_type=jnp.float32)
    q = jnp.dot(x * x, sel, preferred_element_type=jnp.float32)
    fm = 0.5 * jnp.sum(s * s - q, axis=1, keepdims=True)
    h = jnp.maximum(
        jnp.dot(x, w1_ref[...], preferred_element_type=jnp.float32)
        + b1_ref[...], 0.0)
    h = jnp.maximum(
        jnp.dot(h, w2_ref[...], preferred_element_type=jnp.float32)
        + b2_ref[...], 0.0)
    d = jnp.sum(h * w3_ref[...], axis=1, keepdims=True)
    o_ref[...] = d + fm + lin + bias_ref[...]


_tc_mlp = pl.pallas_call(
    _tc_body,
    grid=(BATCH // _BLK,),
    in_specs=[
        pl.BlockSpec((_BLK, DEEP_IN), lambda i: (i, 0)),
        pl.BlockSpec((_BLK, NUM_FIELDS), lambda i: (i, 0)),
        pl.BlockSpec((DEEP_IN, H1), lambda i: (0, 0)),
        pl.BlockSpec((1, H1), lambda i: (0, 0)),
        pl.BlockSpec((H1, H2), lambda i: (0, 0)),
        pl.BlockSpec((1, H2), lambda i: (0, 0)),
        pl.BlockSpec((1, H2), lambda i: (0, 0)),
        pl.BlockSpec((1, 1), lambda i: (0, 0)),
    ],
    out_specs=pl.BlockSpec((_BLK, 1), lambda i: (i, 0)),
    out_shape=jax.ShapeDtypeStruct((BATCH, 1), jnp.float32),
)


def kernel(cat_x, lin_tables, fm_tables, linear_bias, W1, b1, W2, b2, W3, b3):
    cat_flat = cat_x.reshape(-1)
    fm_flat = fm_tables.reshape(NUM_FIELDS * VOCAB, EMBED_DIM)
    lin_flat = lin_tables.reshape(NUM_FIELDS * VOCAB)
    rows, lin_e = _sc_gather(cat_flat, fm_flat, lin_flat)
    deep_in = rows.reshape(BATCH, DEEP_IN)
    bias = (linear_bias + b3).reshape(1, 1)
    return _tc_mlp(deep_in, lin_e.reshape(BATCH, NUM_FIELDS), W1,
                   b1.reshape(1, H1), W2, b2.reshape(1, H2),
                   W3.reshape(1, H2), bias)


# speedup vs baseline: 2.2618x; 2.2618x over previous
import functools

import jax
import jax.numpy as jnp
from jax import lax
from jax.experimental import pallas as pl
from jax.experimental.pallas import tpu as pltpu
from jax.experimental.pallas import tpu_sc as plsc

NUM_FIELDS = 26
VOCAB = 100000
EMBED_DIM = 16
BATCH = 4096
DEEP_IN = NUM_FIELDS * EMBED_DIM
H1, H2 = 128, 64

_NC, _NS, _L = 2, 16, 16
_NW = _NC * _NS
_BPW = BATCH // _NW
_KPF = _BPW // _L


def _sc_body(cat_hbm, fm_hbm, lin_hbm,
             emb_hbm, lin_out_hbm,
             cat_v, base_v, linidx_v, idx_v, vals_v, linv_v, sem):
    wid = lax.axis_index("s") * _NC + lax.axis_index("c")
    base = wid * _BPW

    pltpu.sync_copy(cat_hbm.at[:, pl.ds(base, _BPW)], cat_v)

    for f in range(NUM_FIELDS):
        for k in range(_KPF):
            v = cat_v[f, pl.ds(_L * k, _L)]
            base_v[f, pl.ds(_L * k, _L)] = v + (f * EMBED_DIM * VOCAB)
            linidx_v[f, pl.ds(_L * k, _L)] = v + (f * VOCAB)

    copies = []
    for f in range(NUM_FIELDS):
        copies.append(pltpu.async_copy(
            lin_hbm.at[linidx_v.at[f]], linv_v.at[f], sem))

    for e in range(EMBED_DIM):
        for f in range(NUM_FIELDS):
            for k in range(_KPF):
                idx_v[e, f, pl.ds(_L * k, _L)] = (
                    base_v[f, pl.ds(_L * k, _L)] + (e * VOCAB))
        for f in range(NUM_FIELDS):
            copies.append(pltpu.async_copy(
                fm_hbm.at[idx_v.at[e, f]], vals_v.at[e, f], sem))
    for cp in copies:
        cp.wait()

    pltpu.sync_copy(vals_v, emb_hbm.at[:, :, pl.ds(base, _BPW)])
    pltpu.sync_copy(linv_v, lin_out_hbm.at[:, pl.ds(base, _BPW)])


_sc_gather = functools.partial(
    pl.kernel,
    out_type=[
        jax.ShapeDtypeStruct((EMBED_DIM, NUM_FIELDS, BATCH), jnp.float32),
        jax.ShapeDtypeStruct((NUM_FIELDS, BATCH), jnp.float32),
    ],
    mesh=plsc.VectorSubcoreMesh(core_axis_name="c", subcore_axis_name="s"),
    scratch_types=[
        pltpu.VMEM((NUM_FIELDS, _BPW), jnp.int32),
        pltpu.VMEM((NUM_FIELDS, _BPW), jnp.int32),
        pltpu.VMEM((NUM_FIELDS, _BPW), jnp.int32),
        pltpu.VMEM((EMBED_DIM, NUM_FIELDS, _BPW), jnp.int32),
        pltpu.VMEM((EMBED_DIM, NUM_FIELDS, _BPW), jnp.float32),
        pltpu.VMEM((NUM_FIELDS, _BPW), jnp.float32),
        pltpu.SemaphoreType.DMA,
    ],
    compiler_params=pltpu.CompilerParams(use_tc_tiling_on_sc=False),
)(_sc_body)


_BLK = 512


def _tc_body(x_ref, le_ref, w1_ref, b1_ref, w2_ref, b2_ref, w3_ref, bias_ref,
             o_ref):
    lin = jnp.sum(le_ref[...], axis=0)
    acc = jnp.zeros((_BLK, H1), jnp.float32)
    fm = jnp.zeros((_BLK,), jnp.float32)
    for e in range(EMBED_DIM):
        xe = x_ref[e]
        acc = acc + lax.dot_general(
            xe, w1_ref[e], (((0,), (0,)), ((), ())),
            preferred_element_type=jnp.float32)
        se = jnp.sum(xe, axis=0)
        fm = fm + se * se - jnp.sum(xe * xe, axis=0)
    h = jnp.maximum(acc + b1_ref[...], 0.0)
    h = jnp.maximum(
        jnp.dot(h, w2_ref[...], preferred_element_type=jnp.float32)
        + b2_ref[...], 0.0)
    d = jnp.sum(h * w3_ref[...], axis=1)
    o_ref[...] = (d + 0.5 * fm + lin + bias_ref[0, 0])[:, None]


_tc_mlp = pl.pallas_call(
    _tc_body,
    grid=(BATCH // _BLK,),
    in_specs=[
        pl.BlockSpec((EMBED_DIM, NUM_FIELDS, _BLK), lambda i: (0, 0, i)),
        pl.BlockSpec((NUM_FIELDS, _BLK), lambda i: (0, i)),
        pl.BlockSpec((EMBED_DIM, NUM_FIELDS, H1), lambda i: (0, 0, 0)),
        pl.BlockSpec((1, H1), lambda i: (0, 0)),
        pl.BlockSpec((H1, H2), lambda i: (0, 0)),
        pl.BlockSpec((1, H2), lambda i: (0, 0)),
        pl.BlockSpec((1, H2), lambda i: (0, 0)),
        pl.BlockSpec((1, 1), lambda i: (0, 0)),
    ],
    out_specs=pl.BlockSpec((_BLK, 1), lambda i: (i, 0)),
    out_shape=jax.ShapeDtypeStruct((BATCH, 1), jnp.float32),
)


def kernel(cat_x, lin_tables, fm_tables, linear_bias, W1, b1, W2, b2, W3, b3):
    cat_t = cat_x.T
    fm_flat = jnp.transpose(fm_tables, (0, 2, 1)).reshape(-1)
    lin_flat = lin_tables.reshape(NUM_FIELDS * VOCAB)
    emb, lin_e = _sc_gather(cat_t, fm_flat, lin_flat)
    w1r = W1.reshape(NUM_FIELDS, EMBED_DIM, H1).transpose(1, 0, 2)
    bias = (linear_bias + b3).reshape(1, 1)
    return _tc_mlp(emb, lin_e, w1r, b1.reshape(1, H1), W2,
                   b2.reshape(1, H2), W3.reshape(1, H2), bias)


# speedup vs baseline: 2.2626x; 1.0004x over previous
import functools

import jax
import jax.numpy as jnp
from jax import lax
from jax.experimental import pallas as pl
from jax.experimental.pallas import tpu as pltpu
from jax.experimental.pallas import tpu_sc as plsc

NUM_FIELDS = 26
VOCAB = 100000
EMBED_DIM = 16
BATCH = 4096
DEEP_IN = NUM_FIELDS * EMBED_DIM
H1, H2 = 128, 64

_NC, _NS, _L = 2, 16, 16
_NW = _NC * _NS
_BPW = BATCH // _NW
_KPF = _BPW // _L


def _sc_body(cat_hbm, fm_hbm, lin_hbm,
             emb_hbm, lin_out_hbm,
             cat_v, base_v, linidx_v, idx_v, vals_v, linv_v, sem):
    wid = lax.axis_index("s") * _NC + lax.axis_index("c")
    base = wid * _BPW

    pltpu.sync_copy(cat_hbm.at[:, pl.ds(base, _BPW)], cat_v)

    for f in range(NUM_FIELDS):
        for k in range(_KPF):
            v = cat_v[f, pl.ds(_L * k, _L)]
            base_v[f, pl.ds(_L * k, _L)] = v + (f * EMBED_DIM * VOCAB)
            linidx_v[f, pl.ds(_L * k, _L)] = v + (f * VOCAB)

    copies = []
    for f in range(NUM_FIELDS):
        copies.append(pltpu.async_copy(
            lin_hbm.at[linidx_v.at[f]], linv_v.at[f], sem))

    for e in range(EMBED_DIM):
        for f in range(NUM_FIELDS):
            for k in range(_KPF):
                idx_v[e, f, pl.ds(_L * k, _L)] = (
                    base_v[f, pl.ds(_L * k, _L)] + (e * VOCAB))
        for f in range(NUM_FIELDS):
            copies.append(pltpu.async_copy(
                fm_hbm.at[idx_v.at[e, f]], vals_v.at[e, f], sem))
    for cp in copies:
        cp.wait()

    pltpu.sync_copy(vals_v, emb_hbm.at[:, :, pl.ds(base, _BPW)])
    pltpu.sync_copy(linv_v, lin_out_hbm.at[:, pl.ds(base, _BPW)])


_sc_gather = functools.partial(
    pl.kernel,
    out_type=[
        jax.ShapeDtypeStruct((EMBED_DIM, NUM_FIELDS, BATCH), jnp.float32),
        jax.ShapeDtypeStruct((NUM_FIELDS, BATCH), jnp.float32),
    ],
    mesh=plsc.VectorSubcoreMesh(core_axis_name="c", subcore_axis_name="s"),
    scratch_types=[
        pltpu.VMEM((NUM_FIELDS, _BPW), jnp.int32),
        pltpu.VMEM((NUM_FIELDS, _BPW), jnp.int32),
        pltpu.VMEM((NUM_FIELDS, _BPW), jnp.int32),
        pltpu.VMEM((EMBED_DIM, NUM_FIELDS, _BPW), jnp.int32),
        pltpu.VMEM((EMBED_DIM, NUM_FIELDS, _BPW), jnp.float32),
        pltpu.VMEM((NUM_FIELDS, _BPW), jnp.float32),
        pltpu.SemaphoreType.DMA,
    ],
    compiler_params=pltpu.CompilerParams(use_tc_tiling_on_sc=False),
)(_sc_body)


_BLK = 512


def _tc_body(x_ref, le_ref, w1_ref, b1_ref, w2_ref, b2_ref, w3_ref, bias_ref,
             o_ref):
    lin = jnp.sum(le_ref[...], axis=0)
    acc = jnp.zeros((_BLK, H1), jnp.float32)
    fm = jnp.zeros((_BLK,), jnp.float32)
    for e in range(EMBED_DIM):
        xe = x_ref[e]
        acc = acc + lax.dot_general(
            xe, w1_ref[e], (((0,), (0,)), ((), ())),
            preferred_element_type=jnp.float32)
        se = jnp.sum(xe, axis=0)
        fm = fm + se * se - jnp.sum(xe * xe, axis=0)
    h = jnp.maximum(acc + b1_ref[...], 0.0)
    h = jnp.maximum(
        jnp.dot(h, w2_ref[...], preferred_element_type=jnp.float32)
        + b2_ref[...], 0.0)
    d = jnp.sum(h * w3_ref[...], axis=1)
    o_ref[...] = (d + 0.5 * fm + lin + bias_ref[0, 0])[:, None]


_tc_mlp = pl.pallas_call(
    _tc_body,
    grid=(BATCH // _BLK,),
    in_specs=[
        pl.BlockSpec((EMBED_DIM, NUM_FIELDS, _BLK), lambda i: (0, 0, i)),
        pl.BlockSpec((NUM_FIELDS, _BLK), lambda i: (0, i)),
        pl.BlockSpec((EMBED_DIM, NUM_FIELDS, H1), lambda i: (0, 0, 0)),
        pl.BlockSpec((1, H1), lambda i: (0, 0)),
        pl.BlockSpec((H1, H2), lambda i: (0, 0)),
        pl.BlockSpec((1, H2), lambda i: (0, 0)),
        pl.BlockSpec((1, H2), lambda i: (0, 0)),
        pl.BlockSpec((1, 1), lambda i: (0, 0)),
    ],
    out_specs=pl.BlockSpec((_BLK, 1), lambda i: (i, 0)),
    out_shape=jax.ShapeDtypeStruct((BATCH, 1), jnp.float32),
)


def kernel(cat_x, lin_tables, fm_tables, linear_bias, W1, b1, W2, b2, W3, b3):
    cat_t = cat_x.T
    fm_flat = jnp.transpose(fm_tables, (0, 2, 1)).reshape(-1)
    lin_flat = jnp.transpose(lin_tables, (0, 2, 1)).reshape(NUM_FIELDS * VOCAB)
    emb, lin_e = _sc_gather(cat_t, fm_flat, lin_flat)
    w1r = W1.reshape(NUM_FIELDS, EMBED_DIM, H1).transpose(1, 0, 2)
    bias = (linear_bias + b3).reshape(1, 1)
    return _tc_mlp(emb, lin_e, w1r, b1.reshape(1, H1), W2,
                   b2.reshape(1, H2), W3.reshape(1, H2), bias)
